# baseline (speedup 1.0000x reference)
import functools

import jax
import jax.numpy as jnp
from jax import lax
from jax.experimental import pallas as pl
from jax.experimental.pallas import tpu as pltpu
from jax.experimental.pallas import tpu_sc as plsc

N = 10000
NNZ = 160000
D = 256
HD = 128
NC = 2
NS = 16
P = 10240
RPT = P // NS
CH = 128
NCH = 80
PT = NCH * CH
NNZ_PAD = NS * PT
NCHP = NCH + 2
ZR = 32
NZC = RPT // ZR

R = 1000
GRID = N // R

_MESH = plsc.VectorSubcoreMesh(core_axis_name="c", subcore_axis_name="s")


def _deg_body(sidx, ones_hbm, zeros_hbm, out, idx_c, ones_v, hbuf, obuf, hist):
    cid = lax.axis_index("c")
    sid = lax.axis_index("s")
    pltpu.sync_copy(ones_hbm, ones_v)
    pltpu.sync_copy(zeros_hbm, hbuf)

    def zero_chunk(j, c):
        pltpu.sync_copy(hbuf, hist.at[pl.ds(sid * RPT + j * ZR, ZR)])
        return c

    lax.fori_loop(0, NZC, zero_chunk, 0)
    plsc.subcore_barrier()

    def scatter_chunk(j, c):
        pltpu.sync_copy(sidx.at[1 - cid, sid, j], idx_c)
        pltpu.sync_copy(ones_v, hist.at[idx_c], add=True)
        return c

    lax.fori_loop(0, NCH, scatter_chunk, 0)
    plsc.subcore_barrier()

    def inv_chunk(j, c):
        pltpu.sync_copy(hist.at[pl.ds(sid * RPT + j * ZR, ZR)], hbuf)

        def inv_row(r, c2):
            v = hbuf[r, pl.ds(0, 16)]
            obuf[r] = jnp.where(v > 0.0, 1.0 / v, 0.0)
            return c2

        lax.fori_loop(0, ZR, inv_row, 0)
        pltpu.sync_copy(obuf, out.at[cid, sid, pl.ds(j * ZR, ZR)])
        return c

    lax.fori_loop(0, NZC, inv_chunk, 0)


_deg_call = pl.kernel(
    _deg_body,
    out_type=jax.ShapeDtypeStruct((NC, NS, RPT, 16), jnp.float32),
    mesh=_MESH,
    scratch_types=[
        pltpu.VMEM((CH,), jnp.int32),
        pltpu.VMEM((CH, HD), jnp.float32),
        pltpu.VMEM((ZR, HD), jnp.float32),
        pltpu.VMEM((ZR, 16), jnp.float32),
        pltpu.VMEM_SHARED((P, HD), jnp.float32),
    ],
)


def _hconv_body(x_hbm, gidx, sidx, binv_hbm, dinv_hbm, zeros_hbm,
                out, ef,
                gi_a, si_a, gi_b, si_b, rows_a, rows_b, sbuf, inv_v, acc,
                semi_a, semi_b, semj_a, semj_b, semg_a, semg_b,
                sems_a, sems_b):
    cid = lax.axis_index("c")
    sid = lax.axis_index("s")

    def stage(st, table, inv_hbm, emit):
        pltpu.sync_copy(zeros_hbm, sbuf)

        def zero_chunk(j, c):
            pltpu.async_copy(
                sbuf, acc.at[pl.ds(sid * RPT + j * ZR, ZR)], semg_a)
            return c

        lax.fori_loop(0, NZC, zero_chunk, 0)

        def zero_drain(j, c):
            pltpu.make_async_copy(
                sbuf, acc.at[pl.ds(sid * RPT, ZR)], semg_a).wait()
            return c

        lax.fori_loop(0, NZC, zero_drain, 0)
        plsc.subcore_barrier()

        def start_gi(j, gi, sem):
            pltpu.async_copy(gidx.at[st, cid, sid, j], gi, sem)

        def start_si(j, si, sem):
            pltpu.async_copy(sidx.at[st, sid, j], si, sem)

        def wait_gi(gi, sem):
            pltpu.make_async_copy(gidx.at[st, cid, sid, 0], gi, sem).wait()

        def wait_si(si, sem):
            pltpu.make_async_copy(sidx.at[st, sid, 0], si, sem).wait()

        start_gi(0, gi_a, semi_a)
        start_si(0, si_a, semj_a)
        start_gi(1, gi_b, semi_b)
        start_si(1, si_b, semj_b)

        def pair(t, c):
            a = 2 * t
            wait_gi(gi_a, semi_a)
            ga = pltpu.async_copy(table.at[gi_a], rows_a, semg_a)
            wait_gi(gi_b, semi_b)
            gb = pltpu.async_copy(table.at[gi_b], rows_b, semg_b)
            ga.wait()
            wait_si(si_a, semj_a)
            sa = pltpu.async_copy(rows_a, acc.at[si_a], sems_a, add=True)
            start_gi(a + 2, gi_a, semi_a)
            gb.wait()
            wait_si(si_b, semj_b)
            sb = pltpu.async_copy(rows_b, acc.at[si_b], sems_b, add=True)
            start_gi(a + 3, gi_b, semi_b)
            sa.wait()
            start_si(a + 2, si_a, semj_a)
            sb.wait()
            start_si(a + 3, si_b, semj_b)
            return c

        lax.fori_loop(0, NCH // 2, pair, 0)
        wait_gi(gi_a, semi_a)
        wait_si(si_a, semj_a)
        wait_gi(gi_b, semi_b)
        wait_si(si_b, semj_b)
        plsc.subcore_barrier()

        def scale_chunk(j, c):
            pltpu.sync_copy(acc.at[pl.ds(sid * RPT + j * ZR, ZR)], sbuf)
            pltpu.sync_copy(inv_hbm.at[sid, pl.ds(j * ZR, ZR)], inv_v)

            def scale_row(r, c2):
                s = inv_v[r]
                for k in range(HD // 16):
                    sl = pl.ds(k * 16, 16)
                    sbuf[r, sl] = sbuf[r, sl] * s
                return c2

            lax.fori_loop(0, ZR, scale_row, 0)
            emit(j)
            return c

        lax.fori_loop(0, NZC, scale_chunk, 0)

    def emit_ef(j):
        pltpu.sync_copy(
            sbuf, ef.at[pl.ds(cid * P + sid * RPT + j * ZR, ZR)])

    stage(0, x_hbm, binv_hbm, emit_ef)

    def emit_out(j):
        pltpu.sync_copy(sbuf, out.at[cid, pl.ds(sid * RPT + j * ZR, ZR)])

    stage(1, ef, dinv_hbm, emit_out)


_hconv_call = pl.kernel(
    _hconv_body,
    out_type=(
        jax.ShapeDtypeStruct((NC, P, HD), jnp.float32),
        jax.ShapeDtypeStruct((NC * P, HD), jnp.float32),
    ),
    mesh=_MESH,
    scratch_types=[
        pltpu.VMEM((CH,), jnp.int32),
        pltpu.VMEM((CH,), jnp.int32),
        pltpu.VMEM((CH,), jnp.int32),
        pltpu.VMEM((CH,), jnp.int32),
        pltpu.VMEM((CH, HD), jnp.float32),
        pltpu.VMEM((CH, HD), jnp.float32),
        pltpu.VMEM((ZR, HD), jnp.float32),
        pltpu.VMEM((ZR, 16), jnp.float32),
        pltpu.VMEM_SHARED((P, HD), jnp.float32),
        pltpu.SemaphoreType.DMA,
        pltpu.SemaphoreType.DMA,
        pltpu.SemaphoreType.DMA,
        pltpu.SemaphoreType.DMA,
        pltpu.SemaphoreType.DMA,
        pltpu.SemaphoreType.DMA,
        pltpu.SemaphoreType.DMA,
        pltpu.SemaphoreType.DMA,
    ],
)


def _mm_a_body(feat, tr, wphi, bphi, w1, phi_out, x1s_out):
    ph = jnp.dot(feat[...], wphi[...],
                 preferred_element_type=jnp.float32) + bphi[...]
    phi_out[...] = ph
    xt = tr[...] * ph
    x1 = jnp.dot(xt, w1[...], preferred_element_type=jnp.float32)
    x1s_out[0] = x1[:, :HD]
    x1s_out[1] = x1[:, HD:]


_mm_a_call = pl.pallas_call(
    _mm_a_body,
    grid=(GRID,),
    in_specs=[
        pl.BlockSpec((R, D), lambda i: (i, 0)),
        pl.BlockSpec((R, 1), lambda i: (i, 0)),
        pl.BlockSpec((D, D), lambda i: (0, 0)),
        pl.BlockSpec((1, D), lambda i: (0, 0)),
        pl.BlockSpec((D, D), lambda i: (0, 0)),
    ],
    out_specs=[
        pl.BlockSpec((R, D), lambda i: (i, 0)),
        pl.BlockSpec((NC, R, HD), lambda i: (0, i, 0)),
    ],
    out_shape=[
        jax.ShapeDtypeStruct((N, D), jnp.float32),
        jax.ShapeDtypeStruct((NC, N, HD), jnp.float32),
    ],
)


def _mm_b_body(agg, b1, w2, x2s_out):
    h = jnp.concatenate([agg[0], agg[1]], axis=1) + b1[...]
    rep = jnp.maximum(h, 0.0)
    x2 = jnp.dot(rep, w2[...], preferred_element_type=jnp.float32)
    x2s_out[0] = x2[:, :HD]
    x2s_out[1] = x2[:, HD:]


_mm_b_call = pl.pallas_call(
    _mm_b_body,
    grid=(GRID,),
    in_specs=[
        pl.BlockSpec((NC, R, HD), lambda i: (0, i, 0)),
        pl.BlockSpec((1, D), lambda i: (0, 0)),
        pl.BlockSpec((D, D), lambda i: (0, 0)),
    ],
    out_specs=[
        pl.BlockSpec((NC, R, HD), lambda i: (0, i, 0)),
    ],
    out_shape=[
        jax.ShapeDtypeStruct((NC, N, HD), jnp.float32),
    ],
)


def _mm_c_body(phi, agg, b2, w00b, b00, w10t, w10b, b10, w01, b01, w11, b11,
               y1_out, y0_out):
    rep = jnp.maximum(
        jnp.concatenate([agg[0], agg[1]], axis=1) + b2[...], 0.0)
    t0 = jnp.maximum(
        jnp.dot(rep, w00b[...], preferred_element_type=jnp.float32)
        + b00[...], 0.0)
    y0_out[...] = jnp.dot(t0, w01[...],
                          preferred_element_type=jnp.float32) + b01[...]
    t1 = jnp.maximum(
        jnp.dot(phi[...], w10t[...], preferred_element_type=jnp.float32)
        + jnp.dot(rep, w10b[...], preferred_element_type=jnp.float32)
        + b10[...], 0.0)
    y1_out[...] = jnp.dot(t1, w11[...],
                          preferred_element_type=jnp.float32) + b11[...]


_mm_c_call = pl.pallas_call(
    _mm_c_body,
    grid=(GRID,),
    in_specs=[
        pl.BlockSpec((R, D), lambda i: (i, 0)),
        pl.BlockSpec((NC, R, HD), lambda i: (0, i, 0)),
        pl.BlockSpec((1, D), lambda i: (0, 0)),
        pl.BlockSpec((D, 2 * D), lambda i: (0, 0)),
        pl.BlockSpec((1, 2 * D), lambda i: (0, 0)),
        pl.BlockSpec((D, 2 * D), lambda i: (0, 0)),
        pl.BlockSpec((D, 2 * D), lambda i: (0, 0)),
        pl.BlockSpec((1, 2 * D), lambda i: (0, 0)),
        pl.BlockSpec((2 * D, 1), lambda i: (0, 0)),
        pl.BlockSpec((1, 1), lambda i: (0, 0)),
        pl.BlockSpec((2 * D, 1), lambda i: (0, 0)),
        pl.BlockSpec((1, 1), lambda i: (0, 0)),
    ],
    out_specs=[
        pl.BlockSpec((R, 1), lambda i: (i, 0)),
        pl.BlockSpec((R, 1), lambda i: (i, 0)),
    ],
    out_shape=[
        jax.ShapeDtypeStruct((N, 1), jnp.float32),
        jax.ShapeDtypeStruct((N, 1), jnp.float32),
    ],
)


def kernel(features, treatments, hyperedge_index, W_phi, b_phi, W1, b1,
           W2, b2, W00, b00, W10, b10, W01, b01, W11, b11):
    n_idx = hyperedge_index[0].astype(jnp.int32)
    e_idx = hyperedge_index[1].astype(jnp.int32)
    pad = NNZ_PAD - NNZ
    zpad = jnp.zeros((pad,), jnp.int32)
    dump = jnp.full((pad,), P - 1, jnp.int32)
    n_g = jnp.concatenate([n_idx, zpad])
    e_g = jnp.concatenate([e_idx, zpad])
    n_s = jnp.concatenate([n_idx, dump])
    e_s = jnp.concatenate([e_idx, dump])
    g1 = jnp.stack([n_g, n_g + N])
    g2 = jnp.stack([e_g, e_g + P])
    gidx = jnp.stack([g1, g2]).reshape(2, NC, NS, NCH, CH)
    sidx = jnp.stack([e_s, n_s]).reshape(2, NS, NCH, CH)
    gidx = jnp.concatenate(
        [gidx, jnp.zeros((2, NC, NS, 2, CH), jnp.int32)], axis=3)
    sidx = jnp.concatenate(
        [sidx, jnp.zeros((2, NS, 2, CH), jnp.int32)], axis=2)
    ones_hd = jnp.ones((CH, HD), jnp.float32)
    zhd = jnp.zeros((ZR, HD), jnp.float32)

    deg = _deg_call(sidx, ones_hd, zhd)
    dinv, binv = deg[0], deg[1]

    phi, x1s = _mm_a_call(
        features, treatments.reshape(N, 1), W_phi, b_phi.reshape(1, D), W1)
    agg1, _ = _hconv_call(
        x1s.reshape(NC * N, HD), gidx, sidx, binv, dinv, zhd)
    (x2s,) = _mm_b_call(agg1, b1.reshape(1, D), W2)
    agg2, _ = _hconv_call(
        x2s.reshape(NC * N, HD), gidx, sidx, binv, dinv, zhd)
    y1, y0 = _mm_c_call(
        phi, agg2, b2.reshape(1, D), W00[D:], b00.reshape(1, 2 * D),
        W10[:D], W10[D:], b10.reshape(1, 2 * D), W01, b01.reshape(1, 1),
        W11, b11.reshape(1, 1))
    return (y1.reshape(-1), y0.reshape(-1), phi)


# speedup vs baseline: 1.0279x; 1.0279x over previous
import functools

import jax
import jax.numpy as jnp
from jax import lax
from jax.experimental import pallas as pl
from jax.experimental.pallas import tpu as pltpu
from jax.experimental.pallas import tpu_sc as plsc

N = 10000
NNZ = 160000
D = 256
HD = 128
NC = 2
NS = 16
P = 10240
RPT = P // NS
CH = 128
NCH = 80
PT = NCH * CH
NNZ_PAD = NS * PT
NCHP = NCH + 2
ZR = 32
NZC = RPT // ZR

R = 1000
GRID = N // R

_MESH = plsc.VectorSubcoreMesh(core_axis_name="c", subcore_axis_name="s")


def _deg_body(sidx, ones_hbm, zeros_hbm, out, idx_c, ones_v, hbuf, obuf, hist):
    cid = lax.axis_index("c")
    sid = lax.axis_index("s")
    pltpu.sync_copy(ones_hbm, ones_v)
    pltpu.sync_copy(zeros_hbm, hbuf)

    def zero_chunk(j, c):
        pltpu.sync_copy(hbuf, hist.at[pl.ds(sid * RPT + j * ZR, ZR)])
        return c

    lax.fori_loop(0, NZC, zero_chunk, 0)
    plsc.subcore_barrier()

    def scatter_chunk(j, c):
        pltpu.sync_copy(sidx.at[1 - cid, sid, j], idx_c)
        pltpu.sync_copy(ones_v, hist.at[idx_c], add=True)
        return c

    lax.fori_loop(0, NCH, scatter_chunk, 0)
    plsc.subcore_barrier()

    def inv_chunk(j, c):
        pltpu.sync_copy(hist.at[pl.ds(sid * RPT + j * ZR, ZR)], hbuf)

        def inv_row(r, c2):
            v = hbuf[r, pl.ds(0, 16)]
            obuf[r] = jnp.where(v > 0.0, 1.0 / v, 0.0)
            return c2

        lax.fori_loop(0, ZR, inv_row, 0)
        pltpu.sync_copy(obuf, out.at[cid, sid, pl.ds(j * ZR, ZR)])
        return c

    lax.fori_loop(0, NZC, inv_chunk, 0)


_deg_call = pl.kernel(
    _deg_body,
    out_type=jax.ShapeDtypeStruct((NC, NS, RPT, 16), jnp.float32),
    mesh=_MESH,
    scratch_types=[
        pltpu.VMEM((CH,), jnp.int32),
        pltpu.VMEM((CH, HD), jnp.float32),
        pltpu.VMEM((ZR, HD), jnp.float32),
        pltpu.VMEM((ZR, 16), jnp.float32),
        pltpu.VMEM_SHARED((P, HD), jnp.float32),
    ],
)


def _hconv_body(x_hbm, gidx, sidx, binv_hbm, dinv_hbm, zeros_hbm,
                out, ef,
                gi_a, si_a, gi_b, si_b, rows_a, rows_b, sbuf, inv_v, acc,
                semi_a, semi_b, semj_a, semj_b, semg_a, semg_b,
                sems_a, sems_b):
    cid = lax.axis_index("c")
    sid = lax.axis_index("s")

    def stage(st, table, inv_hbm, emit):
        pltpu.sync_copy(zeros_hbm, sbuf)

        def zero_chunk(j, c):
            pltpu.async_copy(
                sbuf, acc.at[pl.ds(sid * RPT + j * ZR, ZR)], semg_a)
            return c

        lax.fori_loop(0, NZC, zero_chunk, 0)

        def zero_drain(j, c):
            pltpu.make_async_copy(
                sbuf, acc.at[pl.ds(sid * RPT, ZR)], semg_a).wait()
            return c

        lax.fori_loop(0, NZC, zero_drain, 0)
        plsc.subcore_barrier()

        def start_gi(j, gi, sem):
            pltpu.async_copy(gidx.at[st, cid, sid, j], gi, sem)

        def start_si(j, si, sem):
            pltpu.async_copy(sidx.at[st, sid, j], si, sem)

        def wait_gi(gi, sem):
            pltpu.make_async_copy(gidx.at[st, cid, sid, 0], gi, sem).wait()

        def wait_si(si, sem):
            pltpu.make_async_copy(sidx.at[st, sid, 0], si, sem).wait()

        start_gi(0, gi_a, semi_a)
        start_si(0, si_a, semj_a)
        start_gi(1, gi_b, semi_b)
        start_si(1, si_b, semj_b)

        def pair(t, c):
            a = 2 * t
            wait_gi(gi_a, semi_a)
            ga = pltpu.async_copy(table.at[gi_a], rows_a, semg_a)
            wait_gi(gi_b, semi_b)
            gb = pltpu.async_copy(table.at[gi_b], rows_b, semg_b)
            ga.wait()
            wait_si(si_a, semj_a)
            pltpu.sync_copy(rows_a, acc.at[si_a], add=True)
            start_gi(a + 2, gi_a, semi_a)
            start_si(a + 2, si_a, semj_a)
            gb.wait()
            wait_si(si_b, semj_b)
            pltpu.sync_copy(rows_b, acc.at[si_b], add=True)
            start_gi(a + 3, gi_b, semi_b)
            start_si(a + 3, si_b, semj_b)
            return c

        lax.fori_loop(0, NCH // 2, pair, 0)
        wait_gi(gi_a, semi_a)
        wait_si(si_a, semj_a)
        wait_gi(gi_b, semi_b)
        wait_si(si_b, semj_b)
        plsc.subcore_barrier()

        def scale_chunk(j, c):
            pltpu.sync_copy(acc.at[pl.ds(sid * RPT + j * ZR, ZR)], sbuf)
            pltpu.sync_copy(inv_hbm.at[sid, pl.ds(j * ZR, ZR)], inv_v)

            def scale_row(r, c2):
                s = inv_v[r]
                for k in range(HD // 16):
                    sl = pl.ds(k * 16, 16)
                    sbuf[r, sl] = sbuf[r, sl] * s
                return c2

            lax.fori_loop(0, ZR, scale_row, 0)
            emit(j)
            return c

        lax.fori_loop(0, NZC, scale_chunk, 0)

    def emit_ef(j):
        pltpu.sync_copy(
            sbuf, ef.at[pl.ds(cid * P + sid * RPT + j * ZR, ZR)])

    stage(0, x_hbm, binv_hbm, emit_ef)

    def emit_out(j):
        pltpu.sync_copy(sbuf, out.at[cid, pl.ds(sid * RPT + j * ZR, ZR)])

    stage(1, ef, dinv_hbm, emit_out)


_hconv_call = pl.kernel(
    _hconv_body,
    out_type=(
        jax.ShapeDtypeStruct((NC, P, HD), jnp.float32),
        jax.ShapeDtypeStruct((NC * P, HD), jnp.float32),
    ),
    mesh=_MESH,
    scratch_types=[
        pltpu.VMEM((CH,), jnp.int32),
        pltpu.VMEM((CH,), jnp.int32),
        pltpu.VMEM((CH,), jnp.int32),
        pltpu.VMEM((CH,), jnp.int32),
        pltpu.VMEM((CH, HD), jnp.float32),
        pltpu.VMEM((CH, HD), jnp.float32),
        pltpu.VMEM((ZR, HD), jnp.float32),
        pltpu.VMEM((ZR, 16), jnp.float32),
        pltpu.VMEM_SHARED((P, HD), jnp.float32),
        pltpu.SemaphoreType.DMA,
        pltpu.SemaphoreType.DMA,
        pltpu.SemaphoreType.DMA,
        pltpu.SemaphoreType.DMA,
        pltpu.SemaphoreType.DMA,
        pltpu.SemaphoreType.DMA,
        pltpu.SemaphoreType.DMA,
        pltpu.SemaphoreType.DMA,
    ],
)


def _mm_a_body(feat, tr, wphi, bphi, w1, phi_out, x1s_out):
    ph = jnp.dot(feat[...], wphi[...],
                 preferred_element_type=jnp.float32) + bphi[...]
    phi_out[...] = ph
    xt = tr[...] * ph
    x1 = jnp.dot(xt, w1[...], preferred_element_type=jnp.float32)
    x1s_out[0] = x1[:, :HD]
    x1s_out[1] = x1[:, HD:]


_mm_a_call = pl.pallas_call(
    _mm_a_body,
    grid=(GRID,),
    in_specs=[
        pl.BlockSpec((R, D), lambda i: (i, 0)),
        pl.BlockSpec((R, 1), lambda i: (i, 0)),
        pl.BlockSpec((D, D), lambda i: (0, 0)),
        pl.BlockSpec((1, D), lambda i: (0, 0)),
        pl.BlockSpec((D, D), lambda i: (0, 0)),
    ],
    out_specs=[
        pl.BlockSpec((R, D), lambda i: (i, 0)),
        pl.BlockSpec((NC, R, HD), lambda i: (0, i, 0)),
    ],
    out_shape=[
        jax.ShapeDtypeStruct((N, D), jnp.float32),
        jax.ShapeDtypeStruct((NC, N, HD), jnp.float32),
    ],
)


def _mm_b_body(agg, b1, w2, x2s_out):
    h = jnp.concatenate([agg[0], agg[1]], axis=1) + b1[...]
    rep = jnp.maximum(h, 0.0)
    x2 = jnp.dot(rep, w2[...], preferred_element_type=jnp.float32)
    x2s_out[0] = x2[:, :HD]
    x2s_out[1] = x2[:, HD:]


_mm_b_call = pl.pallas_call(
    _mm_b_body,
    grid=(GRID,),
    in_specs=[
        pl.BlockSpec((NC, R, HD), lambda i: (0, i, 0)),
        pl.BlockSpec((1, D), lambda i: (0, 0)),
        pl.BlockSpec((D, D), lambda i: (0, 0)),
    ],
    out_specs=[
        pl.BlockSpec((NC, R, HD), lambda i: (0, i, 0)),
    ],
    out_shape=[
        jax.ShapeDtypeStruct((NC, N, HD), jnp.float32),
    ],
)


def _mm_c_body(phi, agg, b2, w00b, b00, w10t, w10b, b10, w01, b01, w11, b11,
               y1_out, y0_out):
    rep = jnp.maximum(
        jnp.concatenate([agg[0], agg[1]], axis=1) + b2[...], 0.0)
    t0 = jnp.maximum(
        jnp.dot(rep, w00b[...], preferred_element_type=jnp.float32)
        + b00[...], 0.0)
    y0_out[...] = jnp.dot(t0, w01[...],
                          preferred_element_type=jnp.float32) + b01[...]
    t1 = jnp.maximum(
        jnp.dot(phi[...], w10t[...], preferred_element_type=jnp.float32)
        + jnp.dot(rep, w10b[...], preferred_element_type=jnp.float32)
        + b10[...], 0.0)
    y1_out[...] = jnp.dot(t1, w11[...],
                          preferred_element_type=jnp.float32) + b11[...]


_mm_c_call = pl.pallas_call(
    _mm_c_body,
    grid=(GRID,),
    in_specs=[
        pl.BlockSpec((R, D), lambda i: (i, 0)),
        pl.BlockSpec((NC, R, HD), lambda i: (0, i, 0)),
        pl.BlockSpec((1, D), lambda i: (0, 0)),
        pl.BlockSpec((D, 2 * D), lambda i: (0, 0)),
        pl.BlockSpec((1, 2 * D), lambda i: (0, 0)),
        pl.BlockSpec((D, 2 * D), lambda i: (0, 0)),
        pl.BlockSpec((D, 2 * D), lambda i: (0, 0)),
        pl.BlockSpec((1, 2 * D), lambda i: (0, 0)),
        pl.BlockSpec((2 * D, 1), lambda i: (0, 0)),
        pl.BlockSpec((1, 1), lambda i: (0, 0)),
        pl.BlockSpec((2 * D, 1), lambda i: (0, 0)),
        pl.BlockSpec((1, 1), lambda i: (0, 0)),
    ],
    out_specs=[
        pl.BlockSpec((R, 1), lambda i: (i, 0)),
        pl.BlockSpec((R, 1), lambda i: (i, 0)),
    ],
    out_shape=[
        jax.ShapeDtypeStruct((N, 1), jnp.float32),
        jax.ShapeDtypeStruct((N, 1), jnp.float32),
    ],
)


def kernel(features, treatments, hyperedge_index, W_phi, b_phi, W1, b1,
           W2, b2, W00, b00, W10, b10, W01, b01, W11, b11):
    n_idx = hyperedge_index[0].astype(jnp.int32)
    e_idx = hyperedge_index[1].astype(jnp.int32)
    pad = NNZ_PAD - NNZ
    zpad = jnp.zeros((pad,), jnp.int32)
    dump = jnp.full((pad,), P - 1, jnp.int32)
    n_g = jnp.concatenate([n_idx, zpad])
    e_g = jnp.concatenate([e_idx, zpad])
    n_s = jnp.concatenate([n_idx, dump])
    e_s = jnp.concatenate([e_idx, dump])
    g1 = jnp.stack([n_g, n_g + N])
    g2 = jnp.stack([e_g, e_g + P])
    gidx = jnp.stack([g1, g2]).reshape(2, NC, NS, NCH, CH)
    sidx = jnp.stack([e_s, n_s]).reshape(2, NS, NCH, CH)
    gidx = jnp.concatenate(
        [gidx, jnp.zeros((2, NC, NS, 2, CH), jnp.int32)], axis=3)
    sidx = jnp.concatenate(
        [sidx, jnp.zeros((2, NS, 2, CH), jnp.int32)], axis=2)
    ones_hd = jnp.ones((CH, HD), jnp.float32)
    zhd = jnp.zeros((ZR, HD), jnp.float32)

    deg = _deg_call(sidx, ones_hd, zhd)
    dinv, binv = deg[0], deg[1]

    phi, x1s = _mm_a_call(
        features, treatments.reshape(N, 1), W_phi, b_phi.reshape(1, D), W1)
    agg1, _ = _hconv_call(
        x1s.reshape(NC * N, HD), gidx, sidx, binv, dinv, zhd)
    (x2s,) = _mm_b_call(agg1, b1.reshape(1, D), W2)
    agg2, _ = _hconv_call(
        x2s.reshape(NC * N, HD), gidx, sidx, binv, dinv, zhd)
    y1, y0 = _mm_c_call(
        phi, agg2, b2.reshape(1, D), W00[D:], b00.reshape(1, 2 * D),
        W10[:D], W10[D:], b10.reshape(1, 2 * D), W01, b01.reshape(1, 1),
        W11, b11.reshape(1, 1))
    return (y1.reshape(-1), y0.reshape(-1), phi)


# speedup vs baseline: 1.0430x; 1.0146x over previous
import functools

import jax
import jax.numpy as jnp
from jax import lax
from jax.experimental import pallas as pl
from jax.experimental.pallas import tpu as pltpu
from jax.experimental.pallas import tpu_sc as plsc

N = 10000
NNZ = 160000
D = 256
HD = 128
NC = 2
NS = 16
P = 10240
RPT = P // NS
CH = 128
NCH = 80
PT = NCH * CH
NNZ_PAD = NS * PT
NCHP = NCH + 2
ZR = 16
NZC = RPT // ZR

R = 1000
GRID = N // R

_MESH = plsc.VectorSubcoreMesh(core_axis_name="c", subcore_axis_name="s")


def _deg_body(sidx, ones_hbm, zeros_hbm, out, idx_c, ones_v, hbuf, obuf, hist):
    cid = lax.axis_index("c")
    sid = lax.axis_index("s")
    pltpu.sync_copy(ones_hbm, ones_v)
    pltpu.sync_copy(zeros_hbm, hbuf)

    def zero_chunk(j, c):
        pltpu.sync_copy(hbuf, hist.at[pl.ds(sid * RPT + j * ZR, ZR)])
        return c

    lax.fori_loop(0, NZC, zero_chunk, 0)
    plsc.subcore_barrier()

    def scatter_chunk(j, c):
        pltpu.sync_copy(sidx.at[1 - cid, sid, j], idx_c)
        pltpu.sync_copy(ones_v, hist.at[idx_c], add=True)
        return c

    lax.fori_loop(0, NCH, scatter_chunk, 0)
    plsc.subcore_barrier()

    def inv_chunk(j, c):
        pltpu.sync_copy(hist.at[pl.ds(sid * RPT + j * ZR, ZR)], hbuf)

        def inv_row(r, c2):
            v = hbuf[r, pl.ds(0, 16)]
            obuf[r] = jnp.where(v > 0.0, 1.0 / v, 0.0)
            return c2

        lax.fori_loop(0, ZR, inv_row, 0)
        pltpu.sync_copy(obuf, out.at[cid, sid, pl.ds(j * ZR, ZR)])
        return c

    lax.fori_loop(0, NZC, inv_chunk, 0)


_deg_call = pl.kernel(
    _deg_body,
    out_type=jax.ShapeDtypeStruct((NC, NS, RPT, 16), jnp.float32),
    mesh=_MESH,
    scratch_types=[
        pltpu.VMEM((CH,), jnp.int32),
        pltpu.VMEM((CH, HD), jnp.float32),
        pltpu.VMEM((ZR, HD), jnp.float32),
        pltpu.VMEM((ZR, 16), jnp.float32),
        pltpu.VMEM_SHARED((P, HD), jnp.float32),
    ],
)


def _hconv_body(x_hbm, gidx, sidx, binv_hbm, dinv_hbm, zeros_hbm,
                out, ef,
                gi_a, si_a, gi_b, si_b, rows_a, rows_b, sbuf, sbuf_b,
                inv_v, inv_b, acc,
                semi_a, semi_b, semj_a, semj_b, semg_a, semg_b,
                sems_a, sems_b):
    cid = lax.axis_index("c")
    sid = lax.axis_index("s")

    def stage(st, table, inv_hbm, emit):
        pltpu.sync_copy(zeros_hbm, sbuf)

        def zero_chunk(j, c):
            pltpu.async_copy(
                sbuf, acc.at[pl.ds(sid * RPT + j * ZR, ZR)], semg_a)
            return c

        lax.fori_loop(0, NZC, zero_chunk, 0)

        def zero_drain(j, c):
            pltpu.make_async_copy(
                sbuf, acc.at[pl.ds(sid * RPT, ZR)], semg_a).wait()
            return c

        lax.fori_loop(0, NZC, zero_drain, 0)
        plsc.subcore_barrier()

        def start_gi(j, gi, sem):
            pltpu.async_copy(gidx.at[st, cid, sid, j], gi, sem)

        def start_si(j, si, sem):
            pltpu.async_copy(sidx.at[st, sid, j], si, sem)

        def wait_gi(gi, sem):
            pltpu.make_async_copy(gidx.at[st, cid, sid, 0], gi, sem).wait()

        def wait_si(si, sem):
            pltpu.make_async_copy(sidx.at[st, sid, 0], si, sem).wait()

        start_gi(0, gi_a, semi_a)
        start_si(0, si_a, semj_a)
        start_gi(1, gi_b, semi_b)
        start_si(1, si_b, semj_b)

        def pair(t, c):
            a = 2 * t
            wait_gi(gi_a, semi_a)
            ga = pltpu.async_copy(table.at[gi_a], rows_a, semg_a)
            wait_gi(gi_b, semi_b)
            gb = pltpu.async_copy(table.at[gi_b], rows_b, semg_b)
            ga.wait()
            wait_si(si_a, semj_a)
            pltpu.sync_copy(rows_a, acc.at[si_a], add=True)
            start_gi(a + 2, gi_a, semi_a)
            start_si(a + 2, si_a, semj_a)
            gb.wait()
            wait_si(si_b, semj_b)
            pltpu.sync_copy(rows_b, acc.at[si_b], add=True)
            start_gi(a + 3, gi_b, semi_b)
            start_si(a + 3, si_b, semj_b)
            return c

        lax.fori_loop(0, NCH // 2, pair, 0)
        wait_gi(gi_a, semi_a)
        wait_si(si_a, semj_a)
        wait_gi(gi_b, semi_b)
        wait_si(si_b, semj_b)
        plsc.subcore_barrier()

        def start_reads(j, sb, iv, sem_acc, sem_inv):
            jc = jnp.minimum(j, NZC - 1)
            pltpu.async_copy(
                acc.at[pl.ds(sid * RPT + jc * ZR, ZR)], sb, sem_acc)
            pltpu.async_copy(inv_hbm.at[sid, pl.ds(jc * ZR, ZR)], iv, sem_inv)

        def wait_reads(sb, iv, sem_acc, sem_inv):
            pltpu.make_async_copy(
                acc.at[pl.ds(sid * RPT, ZR)], sb, sem_acc).wait()
            pltpu.make_async_copy(
                inv_hbm.at[sid, pl.ds(0, ZR)], iv, sem_inv).wait()

        def scale_rows(sb, iv):
            def scale_row(r, c2):
                s = iv[r]
                for k in range(HD // 16):
                    sl = pl.ds(k * 16, 16)
                    sb[r, sl] = sb[r, sl] * s
                return c2

            lax.fori_loop(0, ZR, scale_row, 0)

        start_reads(0, sbuf, inv_v, semg_a, semi_a)
        start_reads(1, sbuf_b, inv_b, semg_b, semi_b)

        def scale_pair(t, c):
            u = 2 * t
            wait_reads(sbuf, inv_v, semg_a, semi_a)
            scale_rows(sbuf, inv_v)
            ea = emit(u, sbuf, sems_a)
            wait_reads(sbuf_b, inv_b, semg_b, semi_b)
            scale_rows(sbuf_b, inv_b)
            eb = emit(u + 1, sbuf_b, sems_b)
            ea.wait()
            start_reads(u + 2, sbuf, inv_v, semg_a, semi_a)
            eb.wait()
            start_reads(u + 3, sbuf_b, inv_b, semg_b, semi_b)
            return c

        lax.fori_loop(0, NZC // 2, scale_pair, 0)
        wait_reads(sbuf, inv_v, semg_a, semi_a)
        wait_reads(sbuf_b, inv_b, semg_b, semi_b)

    def emit_ef(j, sb, sem):
        return pltpu.async_copy(
            sb, ef.at[pl.ds(cid * P + sid * RPT + j * ZR, ZR)], sem)

    stage(0, x_hbm, binv_hbm, emit_ef)

    def emit_out(j, sb, sem):
        return pltpu.async_copy(
            sb, out.at[cid, pl.ds(sid * RPT + j * ZR, ZR)], sem)

    stage(1, ef, dinv_hbm, emit_out)


_hconv_call = pl.kernel(
    _hconv_body,
    out_type=(
        jax.ShapeDtypeStruct((NC, P, HD), jnp.float32),
        jax.ShapeDtypeStruct((NC * P, HD), jnp.float32),
    ),
    mesh=_MESH,
    scratch_types=[
        pltpu.VMEM((CH,), jnp.int32),
        pltpu.VMEM((CH,), jnp.int32),
        pltpu.VMEM((CH,), jnp.int32),
        pltpu.VMEM((CH,), jnp.int32),
        pltpu.VMEM((CH, HD), jnp.float32),
        pltpu.VMEM((CH, HD), jnp.float32),
        pltpu.VMEM((ZR, HD), jnp.float32),
        pltpu.VMEM((ZR, HD), jnp.float32),
        pltpu.VMEM((ZR, 16), jnp.float32),
        pltpu.VMEM((ZR, 16), jnp.float32),
        pltpu.VMEM_SHARED((P, HD), jnp.float32),
        pltpu.SemaphoreType.DMA,
        pltpu.SemaphoreType.DMA,
        pltpu.SemaphoreType.DMA,
        pltpu.SemaphoreType.DMA,
        pltpu.SemaphoreType.DMA,
        pltpu.SemaphoreType.DMA,
        pltpu.SemaphoreType.DMA,
        pltpu.SemaphoreType.DMA,
    ],
)


def _mm_a_body(feat, tr, wphi, bphi, w1, phi_out, x1s_out):
    ph = jnp.dot(feat[...], wphi[...],
                 preferred_element_type=jnp.float32) + bphi[...]
    phi_out[...] = ph
    xt = tr[...] * ph
    x1 = jnp.dot(xt, w1[...], preferred_element_type=jnp.float32)
    x1s_out[0] = x1[:, :HD]
    x1s_out[1] = x1[:, HD:]


_mm_a_call = pl.pallas_call(
    _mm_a_body,
    grid=(GRID,),
    in_specs=[
        pl.BlockSpec((R, D), lambda i: (i, 0)),
        pl.BlockSpec((R, 1), lambda i: (i, 0)),
        pl.BlockSpec((D, D), lambda i: (0, 0)),
        pl.BlockSpec((1, D), lambda i: (0, 0)),
        pl.BlockSpec((D, D), lambda i: (0, 0)),
    ],
    out_specs=[
        pl.BlockSpec((R, D), lambda i: (i, 0)),
        pl.BlockSpec((NC, R, HD), lambda i: (0, i, 0)),
    ],
    out_shape=[
        jax.ShapeDtypeStruct((N, D), jnp.float32),
        jax.ShapeDtypeStruct((NC, N, HD), jnp.float32),
    ],
)


def _mm_b_body(agg, b1, w2, x2s_out):
    h = jnp.concatenate([agg[0], agg[1]], axis=1) + b1[...]
    rep = jnp.maximum(h, 0.0)
    x2 = jnp.dot(rep, w2[...], preferred_element_type=jnp.float32)
    x2s_out[0] = x2[:, :HD]
    x2s_out[1] = x2[:, HD:]


_mm_b_call = pl.pallas_call(
    _mm_b_body,
    grid=(GRID,),
    in_specs=[
        pl.BlockSpec((NC, R, HD), lambda i: (0, i, 0)),
        pl.BlockSpec((1, D), lambda i: (0, 0)),
        pl.BlockSpec((D, D), lambda i: (0, 0)),
    ],
    out_specs=[
        pl.BlockSpec((NC, R, HD), lambda i: (0, i, 0)),
    ],
    out_shape=[
        jax.ShapeDtypeStruct((NC, N, HD), jnp.float32),
    ],
)


def _mm_c_body(phi, agg, b2, w00b, b00, w10t, w10b, b10, w01, b01, w11, b11,
               y1_out, y0_out):
    rep = jnp.maximum(
        jnp.concatenate([agg[0], agg[1]], axis=1) + b2[...], 0.0)
    t0 = jnp.maximum(
        jnp.dot(rep, w00b[...], preferred_element_type=jnp.float32)
        + b00[...], 0.0)
    y0_out[...] = jnp.dot(t0, w01[...],
                          preferred_element_type=jnp.float32) + b01[...]
    t1 = jnp.maximum(
        jnp.dot(phi[...], w10t[...], preferred_element_type=jnp.float32)
        + jnp.dot(rep, w10b[...], preferred_element_type=jnp.float32)
        + b10[...], 0.0)
    y1_out[...] = jnp.dot(t1, w11[...],
                          preferred_element_type=jnp.float32) + b11[...]


_mm_c_call = pl.pallas_call(
    _mm_c_body,
    grid=(GRID,),
    in_specs=[
        pl.BlockSpec((R, D), lambda i: (i, 0)),
        pl.BlockSpec((NC, R, HD), lambda i: (0, i, 0)),
        pl.BlockSpec((1, D), lambda i: (0, 0)),
        pl.BlockSpec((D, 2 * D), lambda i: (0, 0)),
        pl.BlockSpec((1, 2 * D), lambda i: (0, 0)),
        pl.BlockSpec((D, 2 * D), lambda i: (0, 0)),
        pl.BlockSpec((D, 2 * D), lambda i: (0, 0)),
        pl.BlockSpec((1, 2 * D), lambda i: (0, 0)),
        pl.BlockSpec((2 * D, 1), lambda i: (0, 0)),
        pl.BlockSpec((1, 1), lambda i: (0, 0)),
        pl.BlockSpec((2 * D, 1), lambda i: (0, 0)),
        pl.BlockSpec((1, 1), lambda i: (0, 0)),
    ],
    out_specs=[
        pl.BlockSpec((R, 1), lambda i: (i, 0)),
        pl.BlockSpec((R, 1), lambda i: (i, 0)),
    ],
    out_shape=[
        jax.ShapeDtypeStruct((N, 1), jnp.float32),
        jax.ShapeDtypeStruct((N, 1), jnp.float32),
    ],
)


def kernel(features, treatments, hyperedge_index, W_phi, b_phi, W1, b1,
           W2, b2, W00, b00, W10, b10, W01, b01, W11, b11):
    n_idx = hyperedge_index[0].astype(jnp.int32)
    e_idx = hyperedge_index[1].astype(jnp.int32)
    pad = NNZ_PAD - NNZ
    zpad = jnp.zeros((pad,), jnp.int32)
    dump = jnp.full((pad,), P - 1, jnp.int32)
    n_g = jnp.concatenate([n_idx, zpad])
    e_g = jnp.concatenate([e_idx, zpad])
    n_s = jnp.concatenate([n_idx, dump])
    e_s = jnp.concatenate([e_idx, dump])
    g1 = jnp.stack([n_g, n_g + N])
    g2 = jnp.stack([e_g, e_g + P])
    gidx = jnp.stack([g1, g2]).reshape(2, NC, NS, NCH, CH)
    sidx = jnp.stack([e_s, n_s]).reshape(2, NS, NCH, CH)
    gidx = jnp.concatenate(
        [gidx, jnp.zeros((2, NC, NS, 2, CH), jnp.int32)], axis=3)
    sidx = jnp.concatenate(
        [sidx, jnp.zeros((2, NS, 2, CH), jnp.int32)], axis=2)
    ones_hd = jnp.ones((CH, HD), jnp.float32)
    zhd = jnp.zeros((ZR, HD), jnp.float32)

    deg = _deg_call(sidx, ones_hd, zhd)
    dinv, binv = deg[0], deg[1]

    phi, x1s = _mm_a_call(
        features, treatments.reshape(N, 1), W_phi, b_phi.reshape(1, D), W1)
    agg1, _ = _hconv_call(
        x1s.reshape(NC * N, HD), gidx, sidx, binv, dinv, zhd)
    (x2s,) = _mm_b_call(agg1, b1.reshape(1, D), W2)
    agg2, _ = _hconv_call(
        x2s.reshape(NC * N, HD), gidx, sidx, binv, dinv, zhd)
    y1, y0 = _mm_c_call(
        phi, agg2, b2.reshape(1, D), W00[D:], b00.reshape(1, 2 * D),
        W10[:D], W10[D:], b10.reshape(1, 2 * D), W01, b01.reshape(1, 1),
        W11, b11.reshape(1, 1))
    return (y1.reshape(-1), y0.reshape(-1), phi)


# speedup vs baseline: 1.0694x; 1.0253x over previous
import functools

import jax
import jax.numpy as jnp
from jax import lax
from jax.experimental import pallas as pl
from jax.experimental.pallas import tpu as pltpu
from jax.experimental.pallas import tpu_sc as plsc

N = 10000
NNZ = 160000
D = 256
HD = 128
NC = 2
NS = 16
P = 10240
RPT = P // NS
CH = 128
NCH = 80
PT = NCH * CH
NNZ_PAD = NS * PT
NCHP = NCH + 2
ZR = 16
ZRD = 64
NZCD = 640 // ZRD
NZC = RPT // ZR

R = 1000
GRID = N // R

_MESH = plsc.VectorSubcoreMesh(core_axis_name="c", subcore_axis_name="s")


def _deg_body(sidx, ones_hbm, zeros_hbm, out, idx_a, idx_b, ones_v, hbuf,
              obuf, hist, semd_a, semd_b):
    cid = lax.axis_index("c")
    sid = lax.axis_index("s")
    pltpu.sync_copy(ones_hbm, ones_v)
    pltpu.sync_copy(zeros_hbm, hbuf)

    def zero_chunk(j, c):
        pltpu.sync_copy(hbuf, hist.at[pl.ds(sid * RPT + j * ZRD, ZRD)])
        return c

    lax.fori_loop(0, NZCD, zero_chunk, 0)
    plsc.subcore_barrier()

    def start_idx(j, buf, sem):
        pltpu.async_copy(sidx.at[1 - cid, sid, j], buf, sem)

    def wait_idx(buf, sem):
        pltpu.make_async_copy(sidx.at[0, sid, 0], buf, sem).wait()

    start_idx(0, idx_a, semd_a)
    start_idx(1, idx_b, semd_b)

    def scatter_pair(t, c):
        a = 2 * t
        wait_idx(idx_a, semd_a)
        pltpu.sync_copy(ones_v, hist.at[idx_a], add=True)
        start_idx(a + 2, idx_a, semd_a)
        wait_idx(idx_b, semd_b)
        pltpu.sync_copy(ones_v, hist.at[idx_b], add=True)
        start_idx(a + 3, idx_b, semd_b)
        return c

    lax.fori_loop(0, NCH // 2, scatter_pair, 0)
    wait_idx(idx_a, semd_a)
    wait_idx(idx_b, semd_b)
    plsc.subcore_barrier()

    def inv_chunk(j, c):
        pltpu.sync_copy(hist.at[pl.ds(sid * RPT + j * ZRD, ZRD)], hbuf)

        def inv_row(r, c2):
            v = hbuf[r, pl.ds(0, 16)]
            obuf[r] = jnp.where(v > 0.0, 1.0 / v, 0.0)
            return c2

        lax.fori_loop(0, ZRD, inv_row, 0)
        pltpu.sync_copy(obuf, out.at[cid, sid, pl.ds(j * ZRD, ZRD)])
        return c

    lax.fori_loop(0, NZCD, inv_chunk, 0)


_deg_call = pl.kernel(
    _deg_body,
    out_type=jax.ShapeDtypeStruct((NC, NS, RPT, 16), jnp.float32),
    mesh=_MESH,
    scratch_types=[
        pltpu.VMEM((CH,), jnp.int32),
        pltpu.VMEM((CH,), jnp.int32),
        pltpu.VMEM((CH, HD), jnp.float32),
        pltpu.VMEM((ZRD, HD), jnp.float32),
        pltpu.VMEM((ZRD, 16), jnp.float32),
        pltpu.VMEM_SHARED((P, HD), jnp.float32),
        pltpu.SemaphoreType.DMA,
        pltpu.SemaphoreType.DMA,
    ],
)


def _hconv_body(x_hbm, gidx, sidx, binv_hbm, dinv_hbm, zeros_hbm,
                out, ef,
                gi_a, si_a, gi_b, si_b, rows_a, rows_b, sbuf, sbuf_b,
                inv_v, inv_b, acc,
                semi_a, semi_b, semj_a, semj_b, semg_a, semg_b,
                sems_a, sems_b):
    cid = lax.axis_index("c")
    sid = lax.axis_index("s")

    def stage(st, table, inv_hbm, emit):
        pltpu.sync_copy(zeros_hbm, sbuf)

        def zero_chunk(j, c):
            pltpu.async_copy(
                sbuf, acc.at[pl.ds(sid * RPT + j * ZR, ZR)], semg_a)
            return c

        lax.fori_loop(0, NZC, zero_chunk, 0)

        def zero_drain(j, c):
            pltpu.make_async_copy(
                sbuf, acc.at[pl.ds(sid * RPT, ZR)], semg_a).wait()
            return c

        lax.fori_loop(0, NZC, zero_drain, 0)
        plsc.subcore_barrier()

        def start_gi(j, gi, sem):
            pltpu.async_copy(gidx.at[st, cid, sid, j], gi, sem)

        def start_si(j, si, sem):
            pltpu.async_copy(sidx.at[st, sid, j], si, sem)

        def wait_gi(gi, sem):
            pltpu.make_async_copy(gidx.at[st, cid, sid, 0], gi, sem).wait()

        def wait_si(si, sem):
            pltpu.make_async_copy(sidx.at[st, sid, 0], si, sem).wait()

        start_gi(0, gi_a, semi_a)
        start_si(0, si_a, semj_a)
        start_gi(1, gi_b, semi_b)
        start_si(1, si_b, semj_b)

        def pair(t, c):
            a = 2 * t
            wait_gi(gi_a, semi_a)
            ga = pltpu.async_copy(table.at[gi_a], rows_a, semg_a)
            wait_gi(gi_b, semi_b)
            gb = pltpu.async_copy(table.at[gi_b], rows_b, semg_b)
            ga.wait()
            wait_si(si_a, semj_a)
            pltpu.sync_copy(rows_a, acc.at[si_a], add=True)
            start_gi(a + 2, gi_a, semi_a)
            start_si(a + 2, si_a, semj_a)
            gb.wait()
            wait_si(si_b, semj_b)
            pltpu.sync_copy(rows_b, acc.at[si_b], add=True)
            start_gi(a + 3, gi_b, semi_b)
            start_si(a + 3, si_b, semj_b)
            return c

        lax.fori_loop(0, NCH // 2, pair, 0)
        wait_gi(gi_a, semi_a)
        wait_si(si_a, semj_a)
        wait_gi(gi_b, semi_b)
        wait_si(si_b, semj_b)
        plsc.subcore_barrier()

        def start_reads(j, sb, iv, sem_acc, sem_inv):
            jc = jnp.minimum(j, NZC - 1)
            pltpu.async_copy(
                acc.at[pl.ds(sid * RPT + jc * ZR, ZR)], sb, sem_acc)
            pltpu.async_copy(inv_hbm.at[sid, pl.ds(jc * ZR, ZR)], iv, sem_inv)

        def wait_reads(sb, iv, sem_acc, sem_inv):
            pltpu.make_async_copy(
                acc.at[pl.ds(sid * RPT, ZR)], sb, sem_acc).wait()
            pltpu.make_async_copy(
                inv_hbm.at[sid, pl.ds(0, ZR)], iv, sem_inv).wait()

        def scale_rows(sb, iv):
            def scale_row(r, c2):
                s = iv[r]
                for k in range(HD // 16):
                    sl = pl.ds(k * 16, 16)
                    sb[r, sl] = sb[r, sl] * s
                return c2

            lax.fori_loop(0, ZR, scale_row, 0)

        start_reads(0, sbuf, inv_v, semg_a, semi_a)
        start_reads(1, sbuf_b, inv_b, semg_b, semi_b)

        def scale_pair(t, c):
            u = 2 * t
            wait_reads(sbuf, inv_v, semg_a, semi_a)
            scale_rows(sbuf, inv_v)
            ea = emit(u, sbuf, sems_a)
            wait_reads(sbuf_b, inv_b, semg_b, semi_b)
            scale_rows(sbuf_b, inv_b)
            eb = emit(u + 1, sbuf_b, sems_b)
            ea.wait()
            start_reads(u + 2, sbuf, inv_v, semg_a, semi_a)
            eb.wait()
            start_reads(u + 3, sbuf_b, inv_b, semg_b, semi_b)
            return c

        lax.fori_loop(0, NZC // 2, scale_pair, 0)
        wait_reads(sbuf, inv_v, semg_a, semi_a)
        wait_reads(sbuf_b, inv_b, semg_b, semi_b)

    def emit_ef(j, sb, sem):
        return pltpu.async_copy(
            sb, ef.at[pl.ds(cid * P + sid * RPT + j * ZR, ZR)], sem)

    stage(0, x_hbm, binv_hbm, emit_ef)

    def emit_out(j, sb, sem):
        return pltpu.async_copy(
            sb, out.at[cid, pl.ds(sid * RPT + j * ZR, ZR)], sem)

    stage(1, ef, dinv_hbm, emit_out)


_hconv_call = pl.kernel(
    _hconv_body,
    out_type=(
        jax.ShapeDtypeStruct((NC, P, HD), jnp.float32),
        jax.ShapeDtypeStruct((NC * P, HD), jnp.float32),
    ),
    mesh=_MESH,
    scratch_types=[
        pltpu.VMEM((CH,), jnp.int32),
        pltpu.VMEM((CH,), jnp.int32),
        pltpu.VMEM((CH,), jnp.int32),
        pltpu.VMEM((CH,), jnp.int32),
        pltpu.VMEM((CH, HD), jnp.float32),
        pltpu.VMEM((CH, HD), jnp.float32),
        pltpu.VMEM((ZR, HD), jnp.float32),
        pltpu.VMEM((ZR, HD), jnp.float32),
        pltpu.VMEM((ZR, 16), jnp.float32),
        pltpu.VMEM((ZR, 16), jnp.float32),
        pltpu.VMEM_SHARED((P, HD), jnp.float32),
        pltpu.SemaphoreType.DMA,
        pltpu.SemaphoreType.DMA,
        pltpu.SemaphoreType.DMA,
        pltpu.SemaphoreType.DMA,
        pltpu.SemaphoreType.DMA,
        pltpu.SemaphoreType.DMA,
        pltpu.SemaphoreType.DMA,
        pltpu.SemaphoreType.DMA,
    ],
)


def _mm_a_body(feat, tr, wphi, bphi, w1, phi_out, x1s_out):
    ph = jnp.dot(feat[...], wphi[...],
                 preferred_element_type=jnp.float32) + bphi[...]
    phi_out[...] = ph
    xt = tr[...] * ph
    x1 = jnp.dot(xt, w1[...], preferred_element_type=jnp.float32)
    x1s_out[0] = x1[:, :HD]
    x1s_out[1] = x1[:, HD:]


_mm_a_call = pl.pallas_call(
    _mm_a_body,
    grid=(GRID,),
    in_specs=[
        pl.BlockSpec((R, D), lambda i: (i, 0)),
        pl.BlockSpec((R, 1), lambda i: (i, 0)),
        pl.BlockSpec((D, D), lambda i: (0, 0)),
        pl.BlockSpec((1, D), lambda i: (0, 0)),
        pl.BlockSpec((D, D), lambda i: (0, 0)),
    ],
    out_specs=[
        pl.BlockSpec((R, D), lambda i: (i, 0)),
        pl.BlockSpec((NC, R, HD), lambda i: (0, i, 0)),
    ],
    out_shape=[
        jax.ShapeDtypeStruct((N, D), jnp.float32),
        jax.ShapeDtypeStruct((NC, N, HD), jnp.float32),
    ],
)


def _mm_b_body(agg, b1, w2, x2s_out):
    h = jnp.concatenate([agg[0], agg[1]], axis=1) + b1[...]
    rep = jnp.maximum(h, 0.0)
    x2 = jnp.dot(rep, w2[...], preferred_element_type=jnp.float32)
    x2s_out[0] = x2[:, :HD]
    x2s_out[1] = x2[:, HD:]


_mm_b_call = pl.pallas_call(
    _mm_b_body,
    grid=(GRID,),
    in_specs=[
        pl.BlockSpec((NC, R, HD), lambda i: (0, i, 0)),
        pl.BlockSpec((1, D), lambda i: (0, 0)),
        pl.BlockSpec((D, D), lambda i: (0, 0)),
    ],
    out_specs=[
        pl.BlockSpec((NC, R, HD), lambda i: (0, i, 0)),
    ],
    out_shape=[
        jax.ShapeDtypeStruct((NC, N, HD), jnp.float32),
    ],
)


def _mm_c_body(phi, agg, b2, w00b, b00, w10t, w10b, b10, w01, b01, w11, b11,
               y1_out, y0_out):
    rep = jnp.maximum(
        jnp.concatenate([agg[0], agg[1]], axis=1) + b2[...], 0.0)
    t0 = jnp.maximum(
        jnp.dot(rep, w00b[...], preferred_element_type=jnp.float32)
        + b00[...], 0.0)
    y0_out[...] = jnp.dot(t0, w01[...],
                          preferred_element_type=jnp.float32) + b01[...]
    t1 = jnp.maximum(
        jnp.dot(phi[...], w10t[...], preferred_element_type=jnp.float32)
        + jnp.dot(rep, w10b[...], preferred_element_type=jnp.float32)
        + b10[...], 0.0)
    y1_out[...] = jnp.dot(t1, w11[...],
                          preferred_element_type=jnp.float32) + b11[...]


_mm_c_call = pl.pallas_call(
    _mm_c_body,
    grid=(GRID,),
    in_specs=[
        pl.BlockSpec((R, D), lambda i: (i, 0)),
        pl.BlockSpec((NC, R, HD), lambda i: (0, i, 0)),
        pl.BlockSpec((1, D), lambda i: (0, 0)),
        pl.BlockSpec((D, 2 * D), lambda i: (0, 0)),
        pl.BlockSpec((1, 2 * D), lambda i: (0, 0)),
        pl.BlockSpec((D, 2 * D), lambda i: (0, 0)),
        pl.BlockSpec((D, 2 * D), lambda i: (0, 0)),
        pl.BlockSpec((1, 2 * D), lambda i: (0, 0)),
        pl.BlockSpec((2 * D, 1), lambda i: (0, 0)),
        pl.BlockSpec((1, 1), lambda i: (0, 0)),
        pl.BlockSpec((2 * D, 1), lambda i: (0, 0)),
        pl.BlockSpec((1, 1), lambda i: (0, 0)),
    ],
    out_specs=[
        pl.BlockSpec((R, 1), lambda i: (i, 0)),
        pl.BlockSpec((R, 1), lambda i: (i, 0)),
    ],
    out_shape=[
        jax.ShapeDtypeStruct((N, 1), jnp.float32),
        jax.ShapeDtypeStruct((N, 1), jnp.float32),
    ],
)


def kernel(features, treatments, hyperedge_index, W_phi, b_phi, W1, b1,
           W2, b2, W00, b00, W10, b10, W01, b01, W11, b11):
    n_idx = hyperedge_index[0].astype(jnp.int32)
    e_idx = hyperedge_index[1].astype(jnp.int32)
    pad = NNZ_PAD - NNZ
    zpad = jnp.zeros((pad,), jnp.int32)
    dump = jnp.full((pad,), P - 1, jnp.int32)
    n_g = jnp.concatenate([n_idx, zpad])
    e_g = jnp.concatenate([e_idx, zpad])
    n_s = jnp.concatenate([n_idx, dump])
    e_s = jnp.concatenate([e_idx, dump])
    g1 = jnp.stack([n_g, n_g + N])
    g2 = jnp.stack([e_g, e_g + P])
    gidx = jnp.stack([g1, g2]).reshape(2, NC, NS, NCH, CH)
    sidx = jnp.stack([e_s, n_s]).reshape(2, NS, NCH, CH)
    gidx = jnp.concatenate(
        [gidx, jnp.zeros((2, NC, NS, 2, CH), jnp.int32)], axis=3)
    sidx = jnp.concatenate(
        [sidx, jnp.zeros((2, NS, 2, CH), jnp.int32)], axis=2)
    ones_hd = jnp.ones((CH, HD), jnp.float32)
    zhd = jnp.zeros((ZR, HD), jnp.float32)
    zhd_d = jnp.zeros((ZRD, HD), jnp.float32)

    deg = _deg_call(sidx, ones_hd, zhd_d)
    dinv, binv = deg[0], deg[1]

    phi, x1s = _mm_a_call(
        features, treatments.reshape(N, 1), W_phi, b_phi.reshape(1, D), W1)
    agg1, _ = _hconv_call(
        x1s.reshape(NC * N, HD), gidx, sidx, binv, dinv, zhd)
    (x2s,) = _mm_b_call(agg1, b1.reshape(1, D), W2)
    agg2, _ = _hconv_call(
        x2s.reshape(NC * N, HD), gidx, sidx, binv, dinv, zhd)
    y1, y0 = _mm_c_call(
        phi, agg2, b2.reshape(1, D), W00[D:], b00.reshape(1, 2 * D),
        W10[:D], W10[D:], b10.reshape(1, 2 * D), W01, b01.reshape(1, 1),
        W11, b11.reshape(1, 1))
    return (y1.reshape(-1), y0.reshape(-1), phi)
